# baseline (speedup 1.0000x reference)
import functools

import jax
import jax.numpy as jnp
from jax import lax
from jax.experimental import pallas as pl
from jax.experimental.pallas import tpu as pltpu
from jax.experimental.pallas import tpu_sc as plsc

_BATCH = 16384
_N_FIELDS = 26
_OUT_DIM = 32
_TOTAL = _BATCH * _N_FIELDS
_INPUT_DIM = 1000000

_NC = 2
_NS = 16
_NW = _NC * _NS
_PER_W = _TOTAL // _NW
_C = 832
_K = _PER_W // _C
_H = 4
_G = _K // _H

assert _PER_W * _NW == _TOTAL
assert _K * _C == _PER_W
assert _G * _H == _K


@jax.jit
def _sc_gather(idx2d, table4):
    mesh = plsc.VectorSubcoreMesh(core_axis_name="c", subcore_axis_name="s")

    @functools.partial(
        pl.kernel,
        out_type=jax.ShapeDtypeStruct((_TOTAL, _OUT_DIM), jnp.float32),
        mesh=mesh,
        compiler_params=pltpu.CompilerParams(use_tc_tiling_on_sc=False),
        scratch_types=(
            [pltpu.VMEM((_K, _C), jnp.int32), pltpu.VMEM((_H, _C, _OUT_DIM), jnp.float32)]
            + [pltpu.SemaphoreType.DMA] * (2 * _H)
        ),
    )
    def body(idx_hbm, table_hbm, out_hbm, idx_v, rows_v, *sems):
        sem_g = sems[:_H]
        sem_o = sems[_H:]
        table_r = table_hbm
        out_r = out_hbm
        wid = lax.axis_index("s") * _NC + lax.axis_index("c")
        pltpu.sync_copy(idx_hbm.at[pl.ds(wid * _K, _K)], idx_v)
        base = wid * _PER_W

        def fire_gather(g, h):
            pltpu.async_copy(table_r.at[idx_v.at[g]], rows_v.at[h], sem_g[h])

        def wait_gather(h):
            pltpu.make_async_copy(table_r.at[idx_v.at[0]], rows_v.at[h], sem_g[h]).wait()

        def fire_out(g, h):
            pltpu.async_copy(rows_v.at[h], out_r.at[pl.ds(base + g * _C, _C)], sem_o[h])

        def wait_out(h):
            pltpu.make_async_copy(
                rows_v.at[h], out_r.at[pl.ds(base, _C)], sem_o[h]
            ).wait()

        for h in range(_H):
            fire_gather(h, h)

        def step(i, carry):
            g0 = i * _H
            for h in range(_H):
                wait_gather(h)
                fire_out(g0 + h, h)
            for h in range(_H):
                wait_out(h)

                @pl.when(i < _G - 1)
                def _():
                    fire_gather(g0 + h + _H, h)

            return carry

        lax.fori_loop(0, _G, step, 0)

    return body(idx2d, table4)


def kernel(indices, embedding_table):
    idx2d = indices.astype(jnp.int32).reshape(_TOTAL // _C, _C)
    table4 = embedding_table.reshape(_INPUT_DIM // 4, 128)
    table4 = jax.lax.optimization_barrier(table4)
    table_rm = table4.reshape(_INPUT_DIM, _OUT_DIM)
    out = _sc_gather(idx2d, table_rm)
    return out.reshape(_BATCH, _N_FIELDS, _OUT_DIM)


# speedup vs baseline: 1.1289x; 1.1289x over previous
import functools

import jax
import jax.numpy as jnp
from jax import lax
from jax.experimental import pallas as pl
from jax.experimental.pallas import tpu as pltpu
from jax.experimental.pallas import tpu_sc as plsc

_BATCH = 16384
_N_FIELDS = 26
_OUT_DIM = 32
_TOTAL = _BATCH * _N_FIELDS
_INPUT_DIM = 1000000

_NC = 2
_NS = 16
_NW = _NC * _NS
_PER_W = _TOTAL // _NW
_C = 832
_K = _PER_W // _C
_H = 4
_G = _K // _H

_BLK = 8192

assert _PER_W * _NW == _TOTAL
assert _K * _C == _PER_W
assert _G * _H == _K


def _conv_body(x_ref, o_ref):
    x = x_ref[...]
    xt = jnp.swapaxes(x, 0, 1)
    z = xt.reshape(_BLK // 4, 4, _OUT_DIM)
    o_ref[...] = jnp.concatenate([z[:, 0], z[:, 1], z[:, 2], z[:, 3]], axis=1)


def _relayout_table(table):
    table_t = table.T
    grid = pl.cdiv(_INPUT_DIM, _BLK)
    conv = pl.pallas_call(
        _conv_body,
        grid=(grid,),
        in_specs=[pl.BlockSpec((_OUT_DIM, _BLK), lambda i: (0, i))],
        out_specs=pl.BlockSpec((_BLK // 4, 128), lambda i: (i, 0)),
        out_shape=jax.ShapeDtypeStruct((_INPUT_DIM // 4, 128), jnp.float32),
    )(table_t)
    return conv.reshape(_INPUT_DIM, _OUT_DIM)


def _sc_gather(idx2d, table_rm):
    mesh = plsc.VectorSubcoreMesh(core_axis_name="c", subcore_axis_name="s")

    @functools.partial(
        pl.kernel,
        out_type=jax.ShapeDtypeStruct((_TOTAL, _OUT_DIM), jnp.float32),
        mesh=mesh,
        compiler_params=pltpu.CompilerParams(use_tc_tiling_on_sc=False),
        scratch_types=(
            [pltpu.VMEM((_K, _C), jnp.int32), pltpu.VMEM((_H, _C, _OUT_DIM), jnp.float32)]
            + [pltpu.SemaphoreType.DMA] * (2 * _H)
        ),
    )
    def body(idx_hbm, table_hbm, out_hbm, idx_v, rows_v, *sems):
        sem_g = sems[:_H]
        sem_o = sems[_H:]
        wid = lax.axis_index("s") * _NC + lax.axis_index("c")
        pltpu.sync_copy(idx_hbm.at[pl.ds(wid * _K, _K)], idx_v)
        base = wid * _PER_W

        def fire_gather(g, h):
            pltpu.async_copy(table_hbm.at[idx_v.at[g]], rows_v.at[h], sem_g[h])

        def wait_gather(h):
            pltpu.make_async_copy(table_hbm.at[idx_v.at[0]], rows_v.at[h], sem_g[h]).wait()

        def fire_out(g, h):
            pltpu.async_copy(rows_v.at[h], out_hbm.at[pl.ds(base + g * _C, _C)], sem_o[h])

        def wait_out(h):
            pltpu.make_async_copy(
                rows_v.at[h], out_hbm.at[pl.ds(base, _C)], sem_o[h]
            ).wait()

        for h in range(_H):
            fire_gather(h, h)

        def step(i, carry):
            g0 = i * _H
            for h in range(_H):
                wait_gather(h)
                fire_out(g0 + h, h)
            for h in range(_H):
                wait_out(h)

                @pl.when(i < _G - 1)
                def _():
                    fire_gather(g0 + h + _H, h)

            return carry

        lax.fori_loop(0, _G, step, 0)

    return body(idx2d, table_rm)


@jax.jit
def _impl(indices, embedding_table):
    idx2d = indices.astype(jnp.int32).reshape(_TOTAL // _C, _C)
    table_rm = _relayout_table(embedding_table)
    out = _sc_gather(idx2d, table_rm)
    return out.reshape(_BATCH, _N_FIELDS, _OUT_DIM)


def kernel(indices, embedding_table):
    return _impl(indices, embedding_table)
